# Initial kernel scaffold; baseline (speedup 1.0000x reference)
import functools

import jax
import jax.numpy as jnp
from jax import lax
from jax.experimental import pallas as pl
from jax.experimental.pallas import tpu as pltpu
from jax.experimental.pallas import tpu_sc as plsc

_N = 10000
_E = 320000
_H = 128
_B = 64
_LAYERS = 3

_NC = 2
_NS = 16
_NW = _NC * _NS
_EDGES_PER_TILE = _E // _NW
_CHUNK = 80
_NCHUNK = _EDGES_PER_TILE // _CHUNK
_ROWS_PER_TILE = _N // _NS

_PREC = lax.Precision.HIGHEST


def _edge_agg(h, src, dst, zrows):
    mesh = plsc.VectorSubcoreMesh(core_axis_name="c", subcore_axis_name="s")

    @functools.partial(
        pl.kernel,
        out_type=jax.ShapeDtypeStruct((_NC, _N, _H), jnp.float32),
        mesh=mesh,
        scratch_types=[
            pltpu.VMEM((_CHUNK,), jnp.int32),
            pltpu.VMEM((_CHUNK,), jnp.int32),
            pltpu.VMEM((_CHUNK, _H), jnp.float32),
            pltpu.VMEM_SHARED((_N, _H), jnp.float32),
            pltpu.SemaphoreType.DMA,
        ],
    )
    def k(h_hbm, src_hbm, dst_hbm, z_hbm, out_hbm, src_v, dst_v, rows_v,
          agg_sh, sem):
        cid = lax.axis_index("c")
        sid = lax.axis_index("s")
        wid = sid * _NC + cid

        r0 = sid * _ROWS_PER_TILE
        pltpu.sync_copy(z_hbm, agg_sh.at[pl.ds(r0, _ROWS_PER_TILE)])
        plsc.subcore_barrier()

        ebase = wid * _EDGES_PER_TILE

        def body(j, carry):
            off = ebase + j * _CHUNK
            pltpu.sync_copy(src_hbm.at[pl.ds(off, _CHUNK)], src_v)
            pltpu.sync_copy(dst_hbm.at[pl.ds(off, _CHUNK)], dst_v)
            pltpu.async_copy(h_hbm.at[src_v], rows_v, sem).wait()
            pltpu.sync_copy(rows_v, agg_sh.at[dst_v], add=True)
            return carry

        lax.fori_loop(0, _NCHUNK, body, 0)
        plsc.subcore_barrier()

        pltpu.sync_copy(agg_sh.at[pl.ds(r0, _ROWS_PER_TILE)],
                        out_hbm.at[cid, pl.ds(r0, _ROWS_PER_TILE)])

    return k(h, src, dst, zrows)


_R = 1000
_G = _N // _R


def _onehot(batch_blk):
    cols = lax.broadcasted_iota(jnp.int32, (batch_blk.shape[0], _B), 1)
    return (batch_blk == cols).astype(jnp.float32)


def _leaky(t):
    return jnp.where(t >= 0, t, 0.2 * t)


def _proj_kernel(x_ref, w_ref, b_ref, vne_ref, batch_ref, h_ref, cnt_ref):
    i = pl.program_id(0)
    h = jnp.dot(x_ref[...], w_ref[...], precision=_PREC,
                preferred_element_type=jnp.float32)
    h_ref[...] = h + b_ref[...] + vne_ref[...]
    p = _onehot(batch_ref[...])
    c = jnp.sum(p, axis=0)[:, None]

    @pl.when(i == 0)
    def _():
        cnt_ref[...] = c

    @pl.when(i > 0)
    def _():
        cnt_ref[...] = cnt_ref[...] + c


def _proj(x, w, b, vne, batch2):
    return pl.pallas_call(
        _proj_kernel,
        grid=(_G,),
        in_specs=[
            pl.BlockSpec((_R, _H), lambda i: (i, 0)),
            pl.BlockSpec((_H, _H), lambda i: (0, 0)),
            pl.BlockSpec((1, _H), lambda i: (0, 0)),
            pl.BlockSpec((1, _H), lambda i: (0, 0)),
            pl.BlockSpec((_R, 1), lambda i: (i, 0)),
        ],
        out_specs=[
            pl.BlockSpec((_R, _H), lambda i: (i, 0)),
            pl.BlockSpec((_B, 1), lambda i: (0, 0)),
        ],
        out_shape=[
            jax.ShapeDtypeStruct((_N, _H), jnp.float32),
            jax.ShapeDtypeStruct((_B, 1), jnp.float32),
        ],
    )(x, w, b, vne, batch2)


def _layer_kernel(h_ref, a0_ref, a1_ref, batch_ref, w1_ref, b1_ref, g_ref,
                  bb_ref, m_ref, v_ref, w2_ref, b2_ref, hout_ref, seg_ref):
    i = pl.program_id(0)
    t = h_ref[...] + a0_ref[...] + a1_ref[...]
    t = jnp.dot(t, w1_ref[...], precision=_PREC,
                preferred_element_type=jnp.float32) + b1_ref[...]
    t = _leaky(t)
    t = g_ref[...] * (t - m_ref[...]) * lax.rsqrt(v_ref[...] + 1e-5) + bb_ref[...]
    t = jnp.dot(t, w2_ref[...], precision=_PREC,
                preferred_element_type=jnp.float32) + b2_ref[...]
    t = _leaky(t)
    hout_ref[...] = t
    p = _onehot(batch_ref[...])
    contrib = jnp.dot(p.T, t, precision=_PREC,
                      preferred_element_type=jnp.float32)

    @pl.when(i == 0)
    def _():
        seg_ref[...] = contrib

    @pl.when(i > 0)
    def _():
        seg_ref[...] = seg_ref[...] + contrib


def _layer(h, a0, a1, batch2, w1, b1, g, bb, m, v, w2, b2):
    full = lambda i: (0, 0)
    blk = lambda i: (i, 0)
    vec = pl.BlockSpec((1, _H), full)
    return pl.pallas_call(
        _layer_kernel,
        grid=(_G,),
        in_specs=[
            pl.BlockSpec((_R, _H), blk),
            pl.BlockSpec((_R, _H), blk),
            pl.BlockSpec((_R, _H), blk),
            pl.BlockSpec((_R, 1), blk),
            pl.BlockSpec((_H, _H), full),
            vec, vec, vec, vec, vec,
            pl.BlockSpec((_H, _H), full),
            vec,
        ],
        out_specs=[
            pl.BlockSpec((_R, _H), blk),
            pl.BlockSpec((_B, _H), full),
        ],
        out_shape=[
            jax.ShapeDtypeStruct((_N, _H), jnp.float32),
            jax.ShapeDtypeStruct((_B, _H), jnp.float32),
        ],
    )(h, a0, a1, batch2, w1, b1, g, bb, m, v, w2, b2)


def _vn_kernel(hout_ref, seg_ref, cnt_ref, vnp_ref, w1_ref, b1_ref, w2_ref,
               b2_ref, batch_ref, hin_ref, vn_ref):
    vn_up = seg_ref[...] / jnp.maximum(cnt_ref[...], 1.0)
    u = jnp.dot(vn_up, w1_ref[...], precision=_PREC,
                preferred_element_type=jnp.float32) + b1_ref[...]
    u = _leaky(u)
    u = jnp.dot(u, w2_ref[...], precision=_PREC,
                preferred_element_type=jnp.float32) + b2_ref[...]
    vn = vnp_ref[...] + u
    vn_ref[...] = vn
    p = _onehot(batch_ref[...])
    hin_ref[...] = hout_ref[...] + jnp.dot(
        p, vn, precision=_PREC, preferred_element_type=jnp.float32)


def _vn_step(hout, seg, cnt, vnp, w1, b1, w2, b2, batch2):
    full = lambda i: (0, 0)
    blk = lambda i: (i, 0)
    vec = pl.BlockSpec((1, _H), full)
    return pl.pallas_call(
        _vn_kernel,
        grid=(_G,),
        in_specs=[
            pl.BlockSpec((_R, _H), blk),
            pl.BlockSpec((_B, _H), full),
            pl.BlockSpec((_B, 1), full),
            pl.BlockSpec((_B, _H), full),
            pl.BlockSpec((_H, _H), full),
            vec,
            pl.BlockSpec((_H, _H), full),
            vec,
            pl.BlockSpec((_R, 1), blk),
        ],
        out_specs=[
            pl.BlockSpec((_R, _H), blk),
            pl.BlockSpec((_B, _H), full),
        ],
        out_shape=[
            jax.ShapeDtypeStruct((_N, _H), jnp.float32),
            jax.ShapeDtypeStruct((_B, _H), jnp.float32),
        ],
    )(hout, seg, cnt, vnp, w1, b1, w2, b2, batch2)


def _final_kernel(seg_ref, g_ref, b_ref, m_ref, v_ref, wfc_ref, bfc_ref,
                  out_ref):
    t = g_ref[...] * (seg_ref[...] - m_ref[...]) * lax.rsqrt(
        v_ref[...] + 1e-5) + b_ref[...]
    out_ref[...] = jnp.dot(t, wfc_ref[...], precision=_PREC,
                           preferred_element_type=jnp.float32) + bfc_ref[...]


def _final(seg, g, b, m, v, wfc, bfc):
    return pl.pallas_call(
        _final_kernel,
        out_shape=jax.ShapeDtypeStruct((_B, wfc.shape[1]), jnp.float32),
    )(seg, g, b, m, v, wfc, bfc)


def kernel(x, edge_index, batch, W_proj, b_proj, conv_W1, conv_b1, bn_g, bn_b,
           bn_m, bn_v, conv_W2, conv_b2, vn_emb, vn_W1, vn_b1, vn_W2, vn_b2,
           fbn_g, fbn_b, fbn_m, fbn_v, W_fc, b_fc):
    src = edge_index[0]
    dst = edge_index[1]
    batch2 = batch.reshape(_N, 1).astype(jnp.int32)
    zrows = jnp.zeros((_ROWS_PER_TILE, _H), jnp.float32)
    row = lambda a: a.reshape(1, -1)

    h_in, cnt = _proj(x, W_proj, row(b_proj), vn_emb, batch2)
    vn = jnp.tile(vn_emb, (_B, 1))
    seg = None
    for l in range(_LAYERS):
        parts = _edge_agg(h_in, src, dst, zrows)
        h, seg = _layer(h_in, parts[0], parts[1], batch2, conv_W1[l],
                        row(conv_b1[l]), row(bn_g[l]), row(bn_b[l]),
                        row(bn_m[l]), row(bn_v[l]), conv_W2[l],
                        row(conv_b2[l]))
        if l < _LAYERS - 1:
            h_in, vn = _vn_step(h, seg, cnt, vn, vn_W1[l], row(vn_b1[l]),
                                vn_W2[l], row(vn_b2[l]), batch2)
    return _final(seg, row(fbn_g), row(fbn_b), row(fbn_m), row(fbn_v),
                  W_fc, row(b_fc))


# speedup vs baseline: 4.2179x; 4.2179x over previous
import functools

import jax
import jax.numpy as jnp
from jax import lax
from jax.experimental import pallas as pl
from jax.experimental.pallas import tpu as pltpu
from jax.experimental.pallas import tpu_sc as plsc

_N = 10000
_E = 320000
_H = 128
_B = 64
_LAYERS = 3

_NC = 2
_NS = 16
_NW = _NC * _NS
_EDGES_PER_TILE = _E // _NW
_CHUNK = 80
_NCHUNK = _EDGES_PER_TILE // _CHUNK
_NPAD = 10240
_ROWS_PER_TILE = _NPAD // _NS

_PREC = lax.Precision.HIGHEST


def _edge_agg(h, src, dst, zrows):
    mesh = plsc.VectorSubcoreMesh(core_axis_name="c", subcore_axis_name="s",
                                  num_cores=_NC, num_subcores=_NS)

    @functools.partial(
        pl.kernel,
        out_type=jax.ShapeDtypeStruct((_NC, _NPAD, _H), jnp.float32),
        mesh=mesh,
        scratch_types=[
            pltpu.VMEM((_CHUNK,), jnp.int32),
            pltpu.VMEM((_CHUNK,), jnp.int32),
            pltpu.VMEM((_CHUNK, _H), jnp.float32),
            pltpu.VMEM_SHARED((_NPAD, _H), jnp.float32),
            pltpu.SemaphoreType.DMA,
        ],
    )
    def k(h_hbm, src_hbm, dst_hbm, z_hbm, out_hbm, src_v, dst_v, rows_v,
          agg_sh, sem):
        cid = lax.axis_index("c")
        sid = lax.axis_index("s")
        wid = sid * _NC + cid

        r0 = sid * _ROWS_PER_TILE
        pltpu.sync_copy(z_hbm, agg_sh.at[pl.ds(r0, _ROWS_PER_TILE)])
        plsc.subcore_barrier()

        ebase = wid * _EDGES_PER_TILE

        def body(j, carry):
            off = ebase + j * _CHUNK
            pltpu.sync_copy(src_hbm.at[pl.ds(off, _CHUNK)], src_v)
            pltpu.sync_copy(dst_hbm.at[pl.ds(off, _CHUNK)], dst_v)
            pltpu.async_copy(h_hbm.at[src_v], rows_v, sem).wait()
            pltpu.sync_copy(rows_v, agg_sh.at[dst_v], add=True)
            return carry

        lax.fori_loop(0, _NCHUNK, body, 0)
        plsc.subcore_barrier()

        pltpu.sync_copy(agg_sh.at[pl.ds(r0, _ROWS_PER_TILE)],
                        out_hbm.at[cid, pl.ds(r0, _ROWS_PER_TILE)])

    return k(h, src, dst, zrows)


_R = 1000
_G = _N // _R


def _onehot(batch_blk):
    cols = lax.broadcasted_iota(jnp.int32, (batch_blk.shape[0], _B), 1)
    return (batch_blk == cols).astype(jnp.float32)


def _leaky(t):
    return jnp.where(t >= 0, t, 0.2 * t)


def _proj_kernel(x_ref, w_ref, b_ref, vne_ref, batch_ref, h_ref, cnt_ref):
    i = pl.program_id(0)
    h = jnp.dot(x_ref[...], w_ref[...], precision=_PREC,
                preferred_element_type=jnp.float32)
    h_ref[...] = h + b_ref[...] + vne_ref[...]
    p = _onehot(batch_ref[...])
    c = jnp.sum(p, axis=0)[:, None]

    @pl.when(i == 0)
    def _():
        cnt_ref[...] = c

    @pl.when(i > 0)
    def _():
        cnt_ref[...] = cnt_ref[...] + c


def _proj(x, w, b, vne, batch2):
    return pl.pallas_call(
        _proj_kernel,
        grid=(_G,),
        in_specs=[
            pl.BlockSpec((_R, _H), lambda i: (i, 0)),
            pl.BlockSpec((_H, _H), lambda i: (0, 0)),
            pl.BlockSpec((1, _H), lambda i: (0, 0)),
            pl.BlockSpec((1, _H), lambda i: (0, 0)),
            pl.BlockSpec((_R, 1), lambda i: (i, 0)),
        ],
        out_specs=[
            pl.BlockSpec((_R, _H), lambda i: (i, 0)),
            pl.BlockSpec((_B, 1), lambda i: (0, 0)),
        ],
        out_shape=[
            jax.ShapeDtypeStruct((_N, _H), jnp.float32),
            jax.ShapeDtypeStruct((_B, 1), jnp.float32),
        ],
    )(x, w, b, vne, batch2)


def _layer_kernel(h_ref, a0_ref, a1_ref, batch_ref, w1_ref, b1_ref, g_ref,
                  bb_ref, m_ref, v_ref, w2_ref, b2_ref, hout_ref, seg_ref):
    i = pl.program_id(0)
    t = h_ref[...] + a0_ref[...] + a1_ref[...]
    t = jnp.dot(t, w1_ref[...], precision=_PREC,
                preferred_element_type=jnp.float32) + b1_ref[...]
    t = _leaky(t)
    t = g_ref[...] * (t - m_ref[...]) * lax.rsqrt(v_ref[...] + 1e-5) + bb_ref[...]
    t = jnp.dot(t, w2_ref[...], precision=_PREC,
                preferred_element_type=jnp.float32) + b2_ref[...]
    t = _leaky(t)
    hout_ref[...] = t
    p = _onehot(batch_ref[...])
    contrib = jnp.dot(p.T, t, precision=_PREC,
                      preferred_element_type=jnp.float32)

    @pl.when(i == 0)
    def _():
        seg_ref[...] = contrib

    @pl.when(i > 0)
    def _():
        seg_ref[...] = seg_ref[...] + contrib


def _layer(h, a0, a1, batch2, w1, b1, g, bb, m, v, w2, b2):
    full = lambda i: (0, 0)
    blk = lambda i: (i, 0)
    vec = pl.BlockSpec((1, _H), full)
    return pl.pallas_call(
        _layer_kernel,
        grid=(_G,),
        in_specs=[
            pl.BlockSpec((_R, _H), blk),
            pl.BlockSpec((_R, _H), blk),
            pl.BlockSpec((_R, _H), blk),
            pl.BlockSpec((_R, 1), blk),
            pl.BlockSpec((_H, _H), full),
            vec, vec, vec, vec, vec,
            pl.BlockSpec((_H, _H), full),
            vec,
        ],
        out_specs=[
            pl.BlockSpec((_R, _H), blk),
            pl.BlockSpec((_B, _H), full),
        ],
        out_shape=[
            jax.ShapeDtypeStruct((_N, _H), jnp.float32),
            jax.ShapeDtypeStruct((_B, _H), jnp.float32),
        ],
    )(h, a0, a1, batch2, w1, b1, g, bb, m, v, w2, b2)


def _vn_kernel(hout_ref, seg_ref, cnt_ref, vnp_ref, w1_ref, b1_ref, w2_ref,
               b2_ref, batch_ref, hin_ref, vn_ref):
    vn_up = seg_ref[...] / jnp.maximum(cnt_ref[...], 1.0)
    u = jnp.dot(vn_up, w1_ref[...], precision=_PREC,
                preferred_element_type=jnp.float32) + b1_ref[...]
    u = _leaky(u)
    u = jnp.dot(u, w2_ref[...], precision=_PREC,
                preferred_element_type=jnp.float32) + b2_ref[...]
    vn = vnp_ref[...] + u
    vn_ref[...] = vn
    p = _onehot(batch_ref[...])
    hin_ref[...] = hout_ref[...] + jnp.dot(
        p, vn, precision=_PREC, preferred_element_type=jnp.float32)


def _vn_step(hout, seg, cnt, vnp, w1, b1, w2, b2, batch2):
    full = lambda i: (0, 0)
    blk = lambda i: (i, 0)
    vec = pl.BlockSpec((1, _H), full)
    return pl.pallas_call(
        _vn_kernel,
        grid=(_G,),
        in_specs=[
            pl.BlockSpec((_R, _H), blk),
            pl.BlockSpec((_B, _H), full),
            pl.BlockSpec((_B, 1), full),
            pl.BlockSpec((_B, _H), full),
            pl.BlockSpec((_H, _H), full),
            vec,
            pl.BlockSpec((_H, _H), full),
            vec,
            pl.BlockSpec((_R, 1), blk),
        ],
        out_specs=[
            pl.BlockSpec((_R, _H), blk),
            pl.BlockSpec((_B, _H), full),
        ],
        out_shape=[
            jax.ShapeDtypeStruct((_N, _H), jnp.float32),
            jax.ShapeDtypeStruct((_B, _H), jnp.float32),
        ],
    )(hout, seg, cnt, vnp, w1, b1, w2, b2, batch2)


def _final_kernel(seg_ref, g_ref, b_ref, m_ref, v_ref, wfc_ref, bfc_ref,
                  out_ref):
    t = g_ref[...] * (seg_ref[...] - m_ref[...]) * lax.rsqrt(
        v_ref[...] + 1e-5) + b_ref[...]
    out_ref[...] = jnp.dot(t, wfc_ref[...], precision=_PREC,
                           preferred_element_type=jnp.float32) + bfc_ref[...]


def _final(seg, g, b, m, v, wfc, bfc):
    return pl.pallas_call(
        _final_kernel,
        out_shape=jax.ShapeDtypeStruct((_B, wfc.shape[1]), jnp.float32),
    )(seg, g, b, m, v, wfc, bfc)


def kernel(x, edge_index, batch, W_proj, b_proj, conv_W1, conv_b1, bn_g, bn_b,
           bn_m, bn_v, conv_W2, conv_b2, vn_emb, vn_W1, vn_b1, vn_W2, vn_b2,
           fbn_g, fbn_b, fbn_m, fbn_v, W_fc, b_fc):
    src = edge_index[0]
    dst = edge_index[1]
    batch2 = batch.reshape(_N, 1).astype(jnp.int32)
    zrows = jnp.zeros((_ROWS_PER_TILE, _H), jnp.float32)
    row = lambda a: a.reshape(1, -1)

    h_in, cnt = _proj(x, W_proj, row(b_proj), vn_emb, batch2)
    vn = jnp.tile(vn_emb, (_B, 1))
    seg = None
    for l in range(_LAYERS):
        parts = _edge_agg(h_in, src, dst, zrows)
        h, seg = _layer(h_in, parts[0, :_N], parts[1, :_N], batch2, conv_W1[l],
                        row(conv_b1[l]), row(bn_g[l]), row(bn_b[l]),
                        row(bn_m[l]), row(bn_v[l]), conv_W2[l],
                        row(conv_b2[l]))
        if l < _LAYERS - 1:
            h_in, vn = _vn_step(h, seg, cnt, vn, vn_W1[l], row(vn_b1[l]),
                                vn_W2[l], row(vn_b2[l]), batch2)
    return _final(seg, row(fbn_g), row(fbn_b), row(fbn_m), row(fbn_v),
                  W_fc, row(b_fc))
